# baseline (speedup 1.0000x reference)
from math import log2, e as _e, sqrt

import functools

import jax
import jax.numpy as jnp
from jax import lax
from jax.experimental import pallas as pl
from jax.experimental.pallas import tpu as pltpu

_MASK_VALUE = -1e30


def _fused_attn_kernel(x_ref, wqkv_ref, bqkv_ref, wo_ref, bo_ref,
                       y_ref, a_ref, qkv_scr, acc_sc,
                       *, n_heads, d_keys, lt, d_model):
    i = pl.program_id(1)
    H, dk = n_heads, d_keys
    hd = H * dk
    L = x_ref.shape[1]

    @pl.when(i == 0)
    def _project():
        qkv_scr[...] = (
            jnp.dot(x_ref[0], wqkv_ref[...],
                    preferred_element_type=jnp.float32)
            + bqkv_ref[...]
        )

    diag_mask = (lax.broadcasted_iota(jnp.int32, (lt, lt), 1) >
                 lax.broadcasted_iota(jnp.int32, (lt, lt), 0))

    def _attend(iv):
        width = (iv + 1) * lt
        pre = iv * lt
        q_all = qkv_scr[iv * lt:(iv + 1) * lt, 0:hd]
        for h in range(H):
            q = q_all[:, h * dk:(h + 1) * dk]
            kd = qkv_scr[pre:width, hd + h * dk: hd + (h + 1) * dk]
            sd = lax.dot_general(q, kd, (((1,), (1,)), ((), ())),
                                 preferred_element_type=jnp.float32)
            sd = jnp.where(diag_mask, _MASK_VALUE, sd)
            md = jnp.max(sd, axis=-1, keepdims=True)
            if pre > 0:
                kp = qkv_scr[0:pre, hd + h * dk: hd + (h + 1) * dk]
                sp = lax.dot_general(q, kp, (((1,), (1,)), ((), ())),
                                     preferred_element_type=jnp.float32)
                m = jnp.maximum(jnp.max(sp, axis=-1, keepdims=True), md)
                pp = jnp.exp2(sp - m)
                pd = jnp.exp2(sd - m)
                denom = (jnp.sum(pp, axis=-1, keepdims=True)
                         + jnp.sum(pd, axis=-1, keepdims=True))
                inv = 1.0 / denom
                ap = pp * inv
                ad = pd * inv
                a_ref[0, h, :, 0:pre] = ap.astype(a_ref.dtype)
                a_ref[0, h, :, pre:width] = ad.astype(a_ref.dtype)
                vp = qkv_scr[0:pre, 2 * hd + h * dk: 2 * hd + (h + 1) * dk]
                vd = qkv_scr[pre:width, 2 * hd + h * dk: 2 * hd + (h + 1) * dk]
                acc_sc[:, h * dk:(h + 1) * dk] = (
                    lax.dot_general(ap, vp, (((1,), (0,)), ((), ())),
                                    preferred_element_type=jnp.float32)
                    + lax.dot_general(ad, vd, (((1,), (0,)), ((), ())),
                                      preferred_element_type=jnp.float32))
            else:
                pd = jnp.exp2(sd - md)
                denom = jnp.sum(pd, axis=-1, keepdims=True)
                ad = pd * (1.0 / denom)
                a_ref[0, h, :, 0:width] = ad.astype(a_ref.dtype)
                vd = qkv_scr[pre:width, 2 * hd + h * dk: 2 * hd + (h + 1) * dk]
                acc_sc[:, h * dk:(h + 1) * dk] = lax.dot_general(
                    ad, vd, (((1,), (0,)), ((), ())),
                    preferred_element_type=jnp.float32)
            if width < L:
                a_ref[0, h, :, width:L] = jnp.zeros((lt, L - width),
                                                    a_ref.dtype)
        y_ref[0] = (jnp.dot(acc_sc[...], wo_ref[...],
                            preferred_element_type=jnp.float32)
                    + bo_ref[...]).astype(y_ref.dtype)

    nc = L // lt
    if nc == 1:
        _attend(0)
    else:
        for iv in range(nc):
            @pl.when(i == iv)
            def _(iv=iv):
                _attend(iv)


def kernel(x, wqkv3, bqkv3, wo3, bo):
    B, L, d_model = x.shape
    G, _, dk = wqkv3.shape
    H = G // 3
    hd = H * dk
    lt = 128 if L % 128 == 0 else L
    scale = log2(_e) / sqrt(dk)

    wqkv_flat = jnp.transpose(wqkv3, (1, 0, 2)).reshape(d_model, G * dk)
    bqkv_flat = bqkv3.reshape(1, G * dk)
    qscale = jnp.concatenate(
        [jnp.full((1, hd), scale, wqkv_flat.dtype),
         jnp.ones((1, 2 * hd), wqkv_flat.dtype)], axis=1)
    wqkv_flat = wqkv_flat * qscale
    bqkv_flat = bqkv_flat * qscale
    wo_flat = wo3.reshape(hd, d_model)

    kern = functools.partial(_fused_attn_kernel, n_heads=H, d_keys=dk,
                             lt=lt, d_model=d_model)
    y, attn = pl.pallas_call(
        kern,
        out_shape=(
            jax.ShapeDtypeStruct((B, L, d_model), x.dtype),
            jax.ShapeDtypeStruct((B, H, L, L), x.dtype),
        ),
        grid_spec=pltpu.PrefetchScalarGridSpec(
            num_scalar_prefetch=0,
            grid=(B, L // lt),
            in_specs=[
                pl.BlockSpec((1, L, d_model), lambda b, i: (b, 0, 0)),
                pl.BlockSpec((d_model, G * dk), lambda b, i: (0, 0)),
                pl.BlockSpec((1, G * dk), lambda b, i: (0, 0)),
                pl.BlockSpec((hd, d_model), lambda b, i: (0, 0)),
                pl.BlockSpec((1, d_model), lambda b, i: (0, 0)),
            ],
            out_specs=(
                pl.BlockSpec((1, lt, d_model), lambda b, i: (b, i, 0)),
                pl.BlockSpec((1, H, lt, L), lambda b, i: (b, 0, i, 0)),
            ),
            scratch_shapes=[
                pltpu.VMEM((L, G * dk), jnp.float32),
                pltpu.VMEM((lt, hd), jnp.float32),
            ],
        ),
        compiler_params=pltpu.CompilerParams(
            dimension_semantics=("parallel", "arbitrary"),
            vmem_limit_bytes=60 * 1024 * 1024,
        ),
    )(x, wqkv_flat, bqkv_flat, wo_flat, bo)
    return y, attn


# speedup vs baseline: 1.5631x; 1.5631x over previous
from math import log2, e as _e, sqrt

import functools

import jax
import jax.numpy as jnp
from jax import lax
from jax.experimental import pallas as pl
from jax.experimental.pallas import tpu as pltpu

_MASK_VALUE = -1e30


def _fused_attn_kernel(x_ref, wqkv_ref, bqkv_ref, wo_ref, bo_ref,
                       y_ref, a_ref, qkv_scr, acc_sc,
                       *, n_heads, d_keys, lt, d_model):
    i = pl.program_id(1)
    H, dk = n_heads, d_keys
    hd = H * dk
    L = x_ref.shape[1]

    @pl.when(i == 0)
    def _project():
        qkv_scr[...] = (
            jnp.dot(x_ref[0], wqkv_ref[...],
                    preferred_element_type=jnp.float32)
            + bqkv_ref[...]
        )

    diag_mask = (lax.broadcasted_iota(jnp.int32, (lt, lt), 1) >
                 lax.broadcasted_iota(jnp.int32, (lt, lt), 0))

    def _attend(iv):
        width = (iv + 1) * lt
        pre = iv * lt
        q_all = qkv_scr[iv * lt:(iv + 1) * lt, 0:hd]
        for h in range(H):
            q = q_all[:, h * dk:(h + 1) * dk]
            kd = qkv_scr[pre:width, hd + h * dk: hd + (h + 1) * dk]
            sd = lax.dot_general(q, kd, (((1,), (1,)), ((), ())),
                                 preferred_element_type=jnp.float32)
            sd = jnp.where(diag_mask, _MASK_VALUE, sd)
            md = jnp.max(sd, axis=-1, keepdims=True)
            if pre > 0:
                kp = qkv_scr[0:pre, hd + h * dk: hd + (h + 1) * dk]
                sp = lax.dot_general(q, kp, (((1,), (1,)), ((), ())),
                                     preferred_element_type=jnp.float32)
                m = jnp.maximum(jnp.max(sp, axis=-1, keepdims=True), md)
                pp = jnp.exp2(sp - m)
                pd = jnp.exp2(sd - m)
                denom = (jnp.sum(pp, axis=-1, keepdims=True)
                         + jnp.sum(pd, axis=-1, keepdims=True))
                inv = 1.0 / denom
                ap = pp * inv
                ad = pd * inv
                a_ref[0, h, :, 0:pre] = ap.astype(a_ref.dtype)
                a_ref[0, h, :, pre:width] = ad.astype(a_ref.dtype)
                vp = qkv_scr[0:pre, 2 * hd + h * dk: 2 * hd + (h + 1) * dk]
                vd = qkv_scr[pre:width, 2 * hd + h * dk: 2 * hd + (h + 1) * dk]
                acc_sc[:, h * dk:(h + 1) * dk] = (
                    lax.dot_general(ap, vp, (((1,), (0,)), ((), ())),
                                    preferred_element_type=jnp.float32)
                    + lax.dot_general(ad, vd, (((1,), (0,)), ((), ())),
                                      preferred_element_type=jnp.float32))
            else:
                pd = jnp.exp2(sd - md)
                denom = jnp.sum(pd, axis=-1, keepdims=True)
                ad = pd * (1.0 / denom)
                a_ref[0, h, :, 0:width] = ad.astype(a_ref.dtype)
                vd = qkv_scr[pre:width, 2 * hd + h * dk: 2 * hd + (h + 1) * dk]
                acc_sc[:, h * dk:(h + 1) * dk] = lax.dot_general(
                    ad, vd, (((1,), (0,)), ((), ())),
                    preferred_element_type=jnp.float32)
            if width < L:
                a_ref[0, h, :, width:L] = jnp.zeros((lt, L - width),
                                                    a_ref.dtype)
        y_ref[0] = (jnp.dot(acc_sc[...], wo_ref[...],
                            preferred_element_type=jnp.float32)
                    + bo_ref[...]).astype(y_ref.dtype)

    nc = L // lt
    if nc == 1:
        _attend(0)
    else:
        for iv in range(nc):
            @pl.when(i == iv)
            def _(iv=iv):
                _attend(iv)


def kernel(x, wqkv3, bqkv3, wo3, bo):
    B, L, d_model = x.shape
    G, _, dk = wqkv3.shape
    H = G // 3
    hd = H * dk
    if L % 256 == 0:
        lt = 256
    elif L % 128 == 0:
        lt = 128
    else:
        lt = L
    scale = log2(_e) / sqrt(dk)

    wqkv_flat = jnp.transpose(wqkv3, (1, 0, 2)).reshape(d_model, G * dk)
    bqkv_flat = bqkv3.reshape(1, G * dk)
    qscale = jnp.concatenate(
        [jnp.full((1, hd), scale, wqkv_flat.dtype),
         jnp.ones((1, 2 * hd), wqkv_flat.dtype)], axis=1)
    wqkv_flat = wqkv_flat * qscale
    bqkv_flat = bqkv_flat * qscale
    wo_flat = wo3.reshape(hd, d_model)

    kern = functools.partial(_fused_attn_kernel, n_heads=H, d_keys=dk,
                             lt=lt, d_model=d_model)
    y, attn = pl.pallas_call(
        kern,
        out_shape=(
            jax.ShapeDtypeStruct((B, L, d_model), x.dtype),
            jax.ShapeDtypeStruct((B, H, L, L), x.dtype),
        ),
        grid_spec=pltpu.PrefetchScalarGridSpec(
            num_scalar_prefetch=0,
            grid=(B, L // lt),
            in_specs=[
                pl.BlockSpec((1, L, d_model), lambda b, i: (b, 0, 0)),
                pl.BlockSpec((d_model, G * dk), lambda b, i: (0, 0)),
                pl.BlockSpec((1, G * dk), lambda b, i: (0, 0)),
                pl.BlockSpec((hd, d_model), lambda b, i: (0, 0)),
                pl.BlockSpec((1, d_model), lambda b, i: (0, 0)),
            ],
            out_specs=(
                pl.BlockSpec((1, lt, d_model), lambda b, i: (b, i, 0)),
                pl.BlockSpec((1, H, lt, L), lambda b, i: (b, 0, i, 0)),
            ),
            scratch_shapes=[
                pltpu.VMEM((L, G * dk), jnp.float32),
                pltpu.VMEM((lt, hd), jnp.float32),
            ],
        ),
        compiler_params=pltpu.CompilerParams(
            dimension_semantics=("parallel", "arbitrary"),
            vmem_limit_bytes=60 * 1024 * 1024,
        ),
    )(x, wqkv_flat, bqkv_flat, wo_flat, bo)
    return y, attn
